# baseline (speedup 1.0000x reference)
import functools

import jax
import jax.numpy as jnp
from jax import lax
from jax.experimental import pallas as pl
from jax.experimental.pallas import tpu as pltpu
from jax.experimental.pallas import tpu_sc as plsc

E = 16
TOP_K = 2
D = 1024
F = 2048
V = 64
VP = 128
T = 2048
C = 384
TEMP = 2.0

NC = 2
NS = 16
NW = NC * NS

HIGHEST = jax.lax.Precision.HIGHEST


def _make_sc_gather(row_shape, dtype, b, rows_per_chunk):
    b_per_w = b // NW
    assert b % (8 * NW) == 0
    assert b_per_w % rows_per_chunk == 0
    n_chunks = b_per_w // rows_per_chunk
    mesh = plsc.VectorSubcoreMesh(core_axis_name="c", subcore_axis_name="s")

    @functools.partial(
        pl.kernel,
        mesh=mesh,
        out_type=jax.ShapeDtypeStruct((b,) + row_shape, dtype),
        scratch_types=[
            pltpu.VMEM((b_per_w,), jnp.int32),
            pltpu.VMEM((rows_per_chunk,) + row_shape, dtype),
            pltpu.VMEM((rows_per_chunk,) + row_shape, dtype),
            pltpu.SemaphoreType.DMA,
            pltpu.SemaphoreType.DMA,
            pltpu.SemaphoreType.DMA,
        ],
    )
    def gather_kernel(table_hbm, idx_hbm, out_hbm, idx_v, buf0, buf1,
                      gsem, wsem0, wsem1):
        wid = lax.axis_index("s") * NC + lax.axis_index("c")
        base = wid * b_per_w
        pltpu.sync_copy(idx_hbm.at[pl.ds(base, b_per_w)], idx_v)
        bufs = (buf0, buf1)
        wsems = (wsem0, wsem1)
        pending = [None, None]
        for j in range(n_chunks):
            k = j % 2
            if pending[k] is not None:
                pending[k].wait()
            off = j * rows_per_chunk
            pltpu.async_copy(
                table_hbm.at[idx_v.at[pl.ds(off, rows_per_chunk)]],
                bufs[k], gsem,
            ).wait()
            pending[k] = pltpu.async_copy(
                bufs[k], out_hbm.at[pl.ds(base + off, rows_per_chunk)],
                wsems[k],
            )
        for k in range(2):
            if pending[k] is not None:
                pending[k].wait()

    return gather_kernel


def _router_body(tok_ref, emb_ref, wr_ref, br_ref, wlm_ref, blm_ref,
                 stok_ref, gsel_ref, cidx_ref, cmask_ref, xlm_ref,
                 m_ref, g_ref, p_ref):
    emb = emb_ref[...]
    lr = jnp.dot(emb, wr_ref[...], preferred_element_type=jnp.float32)
    lr = lr + br_ref[...]
    llm = jnp.dot(emb, wlm_ref[...], preferred_element_type=jnp.float32)
    llm = llm + blm_ref[...]

    z = lr * (1.0 / TEMP)
    z = z - jnp.max(z, axis=-1, keepdims=True)
    ez = jnp.exp(z)
    probs = ez / jnp.sum(ez, axis=-1, keepdims=True)

    iota_e = lax.broadcasted_iota(jnp.int32, (1, E), 1).astype(jnp.float32)
    v1 = jnp.max(probs, axis=-1, keepdims=True)
    i1v = jnp.min(jnp.where(probs == v1, iota_e, 1e9), axis=-1, keepdims=True)
    oh1v = (iota_e == i1v).astype(jnp.float32)
    probs2 = jnp.where(oh1v > 0, -1.0, probs)
    v2 = jnp.max(probs2, axis=-1, keepdims=True)
    i2v = jnp.min(jnp.where(probs2 == v2, iota_e, 1e9), axis=-1, keepdims=True)
    oh2v = (iota_e == i2v).astype(jnp.float32)

    s = v1 + v2
    mv = oh1v + oh2v
    gv = (v1 / s) * oh1v + (v2 / s) * oh2v

    tok = tok_ref[...]
    iota_v = lax.broadcasted_iota(jnp.int32, (1, V), 1)
    onehot = (tok == iota_v).astype(jnp.float32)
    bmat = jnp.concatenate([llm, mv, gv, i1v, i2v], axis=1)
    ext = jnp.dot(onehot, bmat, precision=HIGHEST,
                  preferred_element_type=jnp.float32)
    xlm_ref[...] = ext[:, :V]
    m_ref[...] = ext[:, V : V + E]
    g_ref[...] = ext[:, V + E : V + 2 * E]
    i1 = ext[:, V + 2 * E : V + 2 * E + 1]
    i2 = ext[:, V + 2 * E + 1 : V + 2 * E + 2]
    oh1 = (iota_e == i1).astype(jnp.float32)
    oh2 = (iota_e == i2).astype(jnp.float32)

    R = 256
    rows = lax.broadcasted_iota(jnp.int32, (R, R), 0)
    cols = lax.broadcasted_iota(jnp.int32, (R, R), 1)
    tri = (rows > cols).astype(jnp.float32)

    def chunk_body(c, base):
        off = pl.multiple_of(c * R, R)
        mc = m_ref[pl.ds(off, R), :]
        p_ref[pl.ds(off, R), :] = (
            jnp.dot(tri, mc, preferred_element_type=jnp.float32) + base
        )
        return base + jnp.sum(mc, axis=0, keepdims=True)

    lax.fori_loop(0, T // R, chunk_body, jnp.zeros((1, E), jnp.float32))

    tokf = tok.astype(jnp.float32)
    iota_c = lax.broadcasted_iota(jnp.int32, (1, C), 1).astype(jnp.float32)
    for e in range(E):
        pcol = p_ref[:, e : e + 1]
        mcol = m_ref[:, e : e + 1]
        gcol = g_ref[:, e : e + 1]
        key = jnp.where(mcol > 0, pcol, jnp.float32(C))
        oh = (key == iota_c).astype(jnp.float32)
        a = jnp.concatenate([tokf, gcol], axis=1)
        res = lax.dot_general(
            oh, a, (((0,), (0,)), ((), ())),
            preferred_element_type=jnp.float32,
        )
        stok_ref[:, e : e + 1] = res[:, 0:1].astype(jnp.int32)
        gsel_ref[:, e : e + 1] = res[:, 1:2]

    p = p_ref[...]
    ps1 = jnp.sum(jnp.where(oh1 > 0, p, 0.0), axis=-1, keepdims=True)
    ps2 = jnp.sum(jnp.where(oh2 > 0, p, 0.0), axis=-1, keepdims=True)
    ok1 = ps1 < C
    ok2 = ps2 < C
    cidx_ref[:, 0:1] = jnp.where(ok1, i1 * C + ps1, 0.0).astype(jnp.int32)
    cidx_ref[:, 1:2] = jnp.where(ok2, i2 * C + ps2, 0.0).astype(jnp.int32)
    cmask_ref[:, 0:1] = ok1.astype(jnp.float32)
    cmask_ref[:, 1:2] = ok2.astype(jnp.float32)


def _router(tok2, embed, wr, br2, wlm, blm2):
    return pl.pallas_call(
        _router_body,
        out_shape=(
            jax.ShapeDtypeStruct((C, E), jnp.int32),
            jax.ShapeDtypeStruct((C, E), jnp.float32),
            jax.ShapeDtypeStruct((T, 2), jnp.int32),
            jax.ShapeDtypeStruct((T, 2), jnp.float32),
            jax.ShapeDtypeStruct((T, V), jnp.float32),
        ),
        scratch_shapes=[
            pltpu.VMEM((T, E), jnp.float32),
            pltpu.VMEM((T, E), jnp.float32),
            pltpu.VMEM((T, E), jnp.float32),
        ],
    )(tok2, embed, wr, br2, wlm, blm2)


FT = 1024
NF = F // FT


def _ffn_body(xg_ref, w1_ref, b1_ref, w2_ref, b2_ref, g_ref, wlm_ref,
              zg_ref, acc_ref):
    f = pl.program_id(1)
    xg = xg_ref[...].astype(jnp.float32)
    h = jnp.dot(xg, w1_ref[0], preferred_element_type=jnp.float32)
    h = jax.nn.gelu(h + b1_ref[0])
    y = jnp.dot(h, w2_ref[0], preferred_element_type=jnp.float32)

    @pl.when(f == 0)
    def _init():
        acc_ref[...] = y

    @pl.when(f > 0)
    def _acc():
        acc_ref[...] = acc_ref[...] + y

    @pl.when(f == NF - 1)
    def _fin():
        ye = (acc_ref[...] + b2_ref[0]) * g_ref[...]
        z = jnp.dot(ye, wlm_ref[...], preferred_element_type=jnp.float32)
        zg_ref[...] = jnp.concatenate(
            [z, jnp.zeros((C, VP - V), jnp.float32)], axis=1
        )


def _ffn(xg, w1, b1, w2, b2, gcolv, wlm):
    return pl.pallas_call(
        _ffn_body,
        grid=(E, NF),
        in_specs=[
            pl.BlockSpec((C, D), lambda e, f: (e, 0)),
            pl.BlockSpec((1, D, FT), lambda e, f: (e, 0, f)),
            pl.BlockSpec((1, 1, FT), lambda e, f: (e, 0, f)),
            pl.BlockSpec((1, FT, D), lambda e, f: (e, f, 0)),
            pl.BlockSpec((1, 1, D), lambda e, f: (e, 0, 0)),
            pl.BlockSpec((C, 1), lambda e, f: (e, 0)),
            pl.BlockSpec((D, V), lambda e, f: (0, 0)),
        ],
        out_specs=pl.BlockSpec((C, VP), lambda e, f: (e, 0)),
        out_shape=jax.ShapeDtypeStruct((E * C, VP), jnp.float32),
        scratch_shapes=[pltpu.VMEM((C, D), jnp.float32)],
    )(xg, w1, b1.reshape(E, 1, F), w2, b2.reshape(E, 1, D), gcolv, wlm)


def _combine_body(xlm_ref, z0_ref, z1_ref, cm_ref, out_ref):
    cm = cm_ref[...]
    out_ref[...] = (
        xlm_ref[...]
        + z0_ref[:, :V] * cm[:, 0:1]
        + z1_ref[:, :V] * cm[:, 1:2]
    )


def _combine(xlm, z01, cmask):
    return pl.pallas_call(
        _combine_body,
        grid=(1,),
        in_specs=[
            pl.BlockSpec((T, V), lambda r: (0, 0)),
            pl.BlockSpec((T, VP), lambda r: (0, 0)),
            pl.BlockSpec((T, VP), lambda r: (1, 0)),
            pl.BlockSpec((T, 2), lambda r: (0, 0)),
        ],
        out_specs=pl.BlockSpec((T, V), lambda r: (0, 0)),
        out_shape=jax.ShapeDtypeStruct((T, V), jnp.float32),
    )(xlm, z01, z01, cmask)


def _lazy(maker):
    cache = []

    def call(table, idx):
        if not cache:
            cache.append(maker())
        return cache[0](table, idx)

    return call


_dispatch_gather = _lazy(
    lambda: _make_sc_gather((D // 2,), jnp.float32, E * C, 48)
)
_combine_gather = _lazy(lambda: _make_sc_gather((VP,), jnp.float32, 2 * T, 64))


def kernel(input_tensor, embed, Wr, br, W1, b1, W2, b2, Wlm, blm):
    tok = input_tensor.astype(jnp.int32)
    embp = lax.bitcast_convert_type(
        embed.astype(jnp.bfloat16).reshape(V, D // 2, 2), jnp.float32
    )
    stok, gsel, cidx, cmask, xlm = _router(
        tok.reshape(T, 1), embed, Wr, br.reshape(1, E), Wlm, blm.reshape(1, V)
    )
    xgp = _dispatch_gather(embp, stok.T.reshape(E * C))
    xg = lax.bitcast_convert_type(xgp, jnp.bfloat16).reshape(E * C, D)
    zg = _ffn(xg, W1, b1, W2, b2, gsel.T.reshape(E * C, 1), Wlm)
    z01 = _combine_gather(zg, cidx.T.reshape(2 * T))
    return _combine(xlm, z01, cmask)


# speedup vs baseline: 2.2393x; 2.2393x over previous
import functools

import jax
import jax.numpy as jnp
from jax import lax
from jax.experimental import pallas as pl
from jax.experimental.pallas import tpu as pltpu
from jax.experimental.pallas import tpu_sc as plsc

E = 16
TOP_K = 2
D = 1024
F = 2048
V = 64
VP = 128
T = 2048
C = 384
TEMP = 2.0

NC = 2
NS = 16
NW = NC * NS

HIGHEST = jax.lax.Precision.HIGHEST


def _make_sc_gather(row_shape, dtype, b, rows_per_chunk):
    b_per_w = b // NW
    assert b % (8 * NW) == 0
    assert b_per_w % rows_per_chunk == 0
    n_chunks = b_per_w // rows_per_chunk
    mesh = plsc.VectorSubcoreMesh(core_axis_name="c", subcore_axis_name="s")

    @functools.partial(
        pl.kernel,
        mesh=mesh,
        out_type=jax.ShapeDtypeStruct((b,) + row_shape, dtype),
        scratch_types=[
            pltpu.VMEM((b_per_w,), jnp.int32),
            pltpu.VMEM((rows_per_chunk,) + row_shape, dtype),
            pltpu.VMEM((rows_per_chunk,) + row_shape, dtype),
            pltpu.SemaphoreType.DMA,
            pltpu.SemaphoreType.DMA,
            pltpu.SemaphoreType.DMA,
        ],
    )
    def gather_kernel(table_hbm, idx_hbm, out_hbm, idx_v, buf0, buf1,
                      gsem, wsem0, wsem1):
        wid = lax.axis_index("s") * NC + lax.axis_index("c")
        base = wid * b_per_w
        pltpu.sync_copy(idx_hbm.at[pl.ds(base, b_per_w)], idx_v)
        bufs = (buf0, buf1)
        wsems = (wsem0, wsem1)
        pending = [None, None]
        for j in range(n_chunks):
            k = j % 2
            if pending[k] is not None:
                pending[k].wait()
            off = j * rows_per_chunk
            pltpu.async_copy(
                table_hbm.at[idx_v.at[pl.ds(off, rows_per_chunk)]],
                bufs[k], gsem,
            ).wait()
            pending[k] = pltpu.async_copy(
                bufs[k], out_hbm.at[pl.ds(base + off, rows_per_chunk)],
                wsems[k],
            )
        for k in range(2):
            if pending[k] is not None:
                pending[k].wait()

    return gather_kernel


def _router_body(tok_ref, emb_ref, wr_ref, br_ref, wlm_ref, blm_ref,
                 stok_ref, gsel_ref, cidx_ref, cmask_ref, xlm_ref,
                 m_ref, g_ref, p_ref):
    emb = emb_ref[...]
    lr = jnp.dot(emb, wr_ref[...], preferred_element_type=jnp.float32)
    lr = lr + br_ref[...]
    llm = jnp.dot(emb, wlm_ref[...], preferred_element_type=jnp.float32)
    llm = llm + blm_ref[...]

    z = lr * (1.0 / TEMP)
    z = z - jnp.max(z, axis=-1, keepdims=True)
    ez = jnp.exp(z)
    probs = ez / jnp.sum(ez, axis=-1, keepdims=True)

    iota_e = lax.broadcasted_iota(jnp.int32, (1, E), 1).astype(jnp.float32)
    v1 = jnp.max(probs, axis=-1, keepdims=True)
    i1v = jnp.min(jnp.where(probs == v1, iota_e, 1e9), axis=-1, keepdims=True)
    oh1v = (iota_e == i1v).astype(jnp.float32)
    probs2 = jnp.where(oh1v > 0, -1.0, probs)
    v2 = jnp.max(probs2, axis=-1, keepdims=True)
    i2v = jnp.min(jnp.where(probs2 == v2, iota_e, 1e9), axis=-1, keepdims=True)
    oh2v = (iota_e == i2v).astype(jnp.float32)

    s = v1 + v2
    mv = oh1v + oh2v
    gv = (v1 / s) * oh1v + (v2 / s) * oh2v

    tok = tok_ref[...]
    iota_v = lax.broadcasted_iota(jnp.int32, (1, V), 1)
    onehot = (tok == iota_v).astype(jnp.float32)
    bmat = jnp.concatenate([llm, mv, gv, i1v, i2v], axis=1)
    ext = jnp.dot(onehot, bmat, precision=HIGHEST,
                  preferred_element_type=jnp.float32)
    xlm_ref[...] = ext[:, :V]
    m_ref[...] = ext[:, V : V + E]
    g_ref[...] = ext[:, V + E : V + 2 * E]
    i1 = ext[:, V + 2 * E : V + 2 * E + 1]
    i2 = ext[:, V + 2 * E + 1 : V + 2 * E + 2]
    oh1 = (iota_e == i1).astype(jnp.float32)
    oh2 = (iota_e == i2).astype(jnp.float32)

    R = 256
    rows = lax.broadcasted_iota(jnp.int32, (R, R), 0)
    cols = lax.broadcasted_iota(jnp.int32, (R, R), 1)
    tri = (rows > cols).astype(jnp.float32)

    def chunk_body(c, base):
        off = pl.multiple_of(c * R, R)
        mc = m_ref[pl.ds(off, R), :]
        p_ref[pl.ds(off, R), :] = (
            jnp.dot(tri, mc, preferred_element_type=jnp.float32) + base
        )
        return base + jnp.sum(mc, axis=0, keepdims=True)

    lax.fori_loop(0, T // R, chunk_body, jnp.zeros((1, E), jnp.float32))

    tokf = tok.astype(jnp.float32)
    iota_c = lax.broadcasted_iota(jnp.int32, (1, C), 1).astype(jnp.float32)
    for e in range(E):
        pcol = p_ref[:, e : e + 1]
        mcol = m_ref[:, e : e + 1]
        gcol = g_ref[:, e : e + 1]
        key = jnp.where(mcol > 0, pcol, jnp.float32(C))
        oh = (key == iota_c).astype(jnp.float32)
        a = jnp.concatenate([tokf, gcol], axis=1)
        res = lax.dot_general(
            oh, a, (((0,), (0,)), ((), ())),
            preferred_element_type=jnp.float32,
        )
        stok_ref[:, e : e + 1] = res[:, 0:1].astype(jnp.int32)
        gsel_ref[:, e : e + 1] = res[:, 1:2]

    p = p_ref[...]
    ps1 = jnp.sum(jnp.where(oh1 > 0, p, 0.0), axis=-1, keepdims=True)
    ps2 = jnp.sum(jnp.where(oh2 > 0, p, 0.0), axis=-1, keepdims=True)
    ok1 = ps1 < C
    ok2 = ps2 < C
    cidx_ref[:, 0:1] = jnp.where(ok1, i1 * C + ps1, 0.0).astype(jnp.int32)
    cidx_ref[:, 1:2] = jnp.where(ok2, i2 * C + ps2, 0.0).astype(jnp.int32)
    cmask_ref[:, 0:1] = ok1.astype(jnp.float32)
    cmask_ref[:, 1:2] = ok2.astype(jnp.float32)


def _router(tok2, embed, wr, br2, wlm, blm2):
    return pl.pallas_call(
        _router_body,
        out_shape=(
            jax.ShapeDtypeStruct((C, E), jnp.int32),
            jax.ShapeDtypeStruct((C, E), jnp.float32),
            jax.ShapeDtypeStruct((T, 2), jnp.int32),
            jax.ShapeDtypeStruct((T, 2), jnp.float32),
            jax.ShapeDtypeStruct((T, V), jnp.float32),
        ),
        scratch_shapes=[
            pltpu.VMEM((T, E), jnp.float32),
            pltpu.VMEM((T, E), jnp.float32),
            pltpu.VMEM((T, E), jnp.float32),
        ],
    )(tok2, embed, wr, br2, wlm, blm2)


FT = 1024
NF = F // FT


def _ffn_body(stok_ref, emb_ref, w1_ref, b1_ref, w2_ref, b2_ref, g_ref,
              wlm_ref, zg_ref, acc_ref):
    f = pl.program_id(1)
    iota_v = lax.broadcasted_iota(jnp.int32, (1, V), 1)
    onehot = (stok_ref[...] == iota_v).astype(jnp.float32)
    xg = jnp.dot(onehot, emb_ref[...], preferred_element_type=jnp.float32)
    h = jnp.dot(xg, w1_ref[0], preferred_element_type=jnp.float32)
    h = jax.nn.gelu(h + b1_ref[0])
    y = jnp.dot(h, w2_ref[0], preferred_element_type=jnp.float32)

    @pl.when(f == 0)
    def _init():
        acc_ref[...] = y

    @pl.when(f > 0)
    def _acc():
        acc_ref[...] = acc_ref[...] + y

    @pl.when(f == NF - 1)
    def _fin():
        ye = (acc_ref[...] + b2_ref[0]) * g_ref[...]
        z = jnp.dot(ye, wlm_ref[...], preferred_element_type=jnp.float32)
        zg_ref[...] = jnp.concatenate(
            [z, jnp.zeros((C, VP - V), jnp.float32)], axis=1
        )


def _ffn(stokcol, emb, w1, b1, w2, b2, gcolv, wlm):
    return pl.pallas_call(
        _ffn_body,
        grid=(E, NF),
        in_specs=[
            pl.BlockSpec((C, 1), lambda e, f: (e, 0)),
            pl.BlockSpec((V, D), lambda e, f: (0, 0)),
            pl.BlockSpec((1, D, FT), lambda e, f: (e, 0, f)),
            pl.BlockSpec((1, 1, FT), lambda e, f: (e, 0, f)),
            pl.BlockSpec((1, FT, D), lambda e, f: (e, f, 0)),
            pl.BlockSpec((1, 1, D), lambda e, f: (e, 0, 0)),
            pl.BlockSpec((C, 1), lambda e, f: (e, 0)),
            pl.BlockSpec((D, V), lambda e, f: (0, 0)),
        ],
        out_specs=pl.BlockSpec((C, VP), lambda e, f: (e, 0)),
        out_shape=jax.ShapeDtypeStruct((E * C, VP), jnp.float32),
        scratch_shapes=[pltpu.VMEM((C, D), jnp.float32)],
    )(stokcol, emb, w1, b1.reshape(E, 1, F), w2, b2.reshape(E, 1, D),
      gcolv, wlm)


def _combine_body(xlm_ref, z0_ref, z1_ref, cm_ref, out_ref):
    cm = cm_ref[...]
    out_ref[...] = (
        xlm_ref[...]
        + z0_ref[:, :V] * cm[:, 0:1]
        + z1_ref[:, :V] * cm[:, 1:2]
    )


def _combine(xlm, z01, cmask):
    return pl.pallas_call(
        _combine_body,
        grid=(1,),
        in_specs=[
            pl.BlockSpec((T, V), lambda r: (0, 0)),
            pl.BlockSpec((T, VP), lambda r: (0, 0)),
            pl.BlockSpec((T, VP), lambda r: (1, 0)),
            pl.BlockSpec((T, 2), lambda r: (0, 0)),
        ],
        out_specs=pl.BlockSpec((T, V), lambda r: (0, 0)),
        out_shape=jax.ShapeDtypeStruct((T, V), jnp.float32),
    )(xlm, z01, z01, cmask)


def _lazy(maker):
    cache = []

    def call(table, idx):
        if not cache:
            cache.append(maker())
        return cache[0](table, idx)

    return call


_combine_gather = _lazy(lambda: _make_sc_gather((VP,), jnp.float32, 2 * T, 64))


def kernel(input_tensor, embed, Wr, br, W1, b1, W2, b2, Wlm, blm):
    tok = input_tensor.astype(jnp.int32)
    stok, gsel, cidx, cmask, xlm = _router(
        tok.reshape(T, 1), embed, Wr, br.reshape(1, E), Wlm, blm.reshape(1, V)
    )
    zg = _ffn(stok.T.reshape(E * C, 1), embed, W1, b1, W2, b2,
              gsel.T.reshape(E * C, 1), Wlm)
    z01 = _combine_gather(zg, cidx.T.reshape(2 * T))
    return _combine(xlm, z01, cmask)


# speedup vs baseline: 2.4400x; 1.0897x over previous
import functools

import jax
import jax.numpy as jnp
from jax import lax
from jax.experimental import pallas as pl
from jax.experimental.pallas import tpu as pltpu
from jax.experimental.pallas import tpu_sc as plsc

E = 16
TOP_K = 2
D = 1024
F = 2048
V = 64
VP = 128
T = 2048
C = 384
TEMP = 2.0

NC = 2
NS = 16
NW = NC * NS

HIGHEST = jax.lax.Precision.HIGHEST


def _make_sc_gather(row_shape, dtype, b, rows_per_chunk):
    b_per_w = b // NW
    assert b % (8 * NW) == 0
    assert b_per_w % rows_per_chunk == 0
    n_chunks = b_per_w // rows_per_chunk
    mesh = plsc.VectorSubcoreMesh(core_axis_name="c", subcore_axis_name="s")

    @functools.partial(
        pl.kernel,
        mesh=mesh,
        out_type=jax.ShapeDtypeStruct((b,) + row_shape, dtype),
        scratch_types=[
            pltpu.VMEM((b_per_w,), jnp.int32),
            pltpu.VMEM((rows_per_chunk,) + row_shape, dtype),
            pltpu.VMEM((rows_per_chunk,) + row_shape, dtype),
            pltpu.SemaphoreType.DMA,
            pltpu.SemaphoreType.DMA,
            pltpu.SemaphoreType.DMA,
        ],
    )
    def gather_kernel(table_hbm, idx_hbm, out_hbm, idx_v, buf0, buf1,
                      gsem, wsem0, wsem1):
        wid = lax.axis_index("s") * NC + lax.axis_index("c")
        base = wid * b_per_w
        pltpu.sync_copy(idx_hbm.at[pl.ds(base, b_per_w)], idx_v)
        bufs = (buf0, buf1)
        wsems = (wsem0, wsem1)
        pending = [None, None]
        for j in range(n_chunks):
            k = j % 2
            if pending[k] is not None:
                pending[k].wait()
            off = j * rows_per_chunk
            pltpu.async_copy(
                table_hbm.at[idx_v.at[pl.ds(off, rows_per_chunk)]],
                bufs[k], gsem,
            ).wait()
            pending[k] = pltpu.async_copy(
                bufs[k], out_hbm.at[pl.ds(base + off, rows_per_chunk)],
                wsems[k],
            )
        for k in range(2):
            if pending[k] is not None:
                pending[k].wait()

    return gather_kernel


def _router_body(tok_ref, emb_ref, wr_ref, br_ref, wlm_ref, blm_ref,
                 stok_ref, gsel_ref, cidx_ref, cmask_ref, xlm_ref,
                 m_ref, g_ref, p_ref):
    emb = emb_ref[...]
    lr = jnp.dot(emb, wr_ref[...], preferred_element_type=jnp.float32)
    lr = lr + br_ref[...]
    llm = jnp.dot(emb, wlm_ref[...], preferred_element_type=jnp.float32)
    llm = llm + blm_ref[...]

    z = lr * (1.0 / TEMP)
    z = z - jnp.max(z, axis=-1, keepdims=True)
    ez = jnp.exp(z)
    probs = ez / jnp.sum(ez, axis=-1, keepdims=True)

    iota_e = lax.broadcasted_iota(jnp.int32, (1, E), 1).astype(jnp.float32)
    v1 = jnp.max(probs, axis=-1, keepdims=True)
    i1v = jnp.min(jnp.where(probs == v1, iota_e, 1e9), axis=-1, keepdims=True)
    oh1v = (iota_e == i1v).astype(jnp.float32)
    probs2 = jnp.where(oh1v > 0, -1.0, probs)
    v2 = jnp.max(probs2, axis=-1, keepdims=True)
    i2v = jnp.min(jnp.where(probs2 == v2, iota_e, 1e9), axis=-1, keepdims=True)
    oh2v = (iota_e == i2v).astype(jnp.float32)

    s = v1 + v2
    mv = oh1v + oh2v
    gv = (v1 / s) * oh1v + (v2 / s) * oh2v

    tok = tok_ref[...]
    iota_v = lax.broadcasted_iota(jnp.int32, (1, V), 1)
    onehot = (tok == iota_v).astype(jnp.float32)
    bmat = jnp.concatenate([llm, mv, gv, i1v, i2v], axis=1)
    ext = jnp.dot(onehot, bmat, precision=HIGHEST,
                  preferred_element_type=jnp.float32)
    xlm_ref[...] = ext[:, :V]
    m_ref[...] = ext[:, V : V + E]
    g_ref[...] = ext[:, V + E : V + 2 * E]
    i1 = ext[:, V + 2 * E : V + 2 * E + 1]
    i2 = ext[:, V + 2 * E + 1 : V + 2 * E + 2]
    oh1 = (iota_e == i1).astype(jnp.float32)
    oh2 = (iota_e == i2).astype(jnp.float32)

    R = 256
    rows = lax.broadcasted_iota(jnp.int32, (R, R), 0)
    cols = lax.broadcasted_iota(jnp.int32, (R, R), 1)
    tri = (rows > cols).astype(jnp.bfloat16)

    def chunk_body(c, base):
        off = pl.multiple_of(c * R, R)
        mc = m_ref[pl.ds(off, R), :].astype(jnp.bfloat16)
        p_ref[pl.ds(off, R), :] = (
            jnp.dot(tri, mc, preferred_element_type=jnp.float32) + base
        )
        return base + jnp.sum(mc.astype(jnp.float32), axis=0, keepdims=True)

    lax.fori_loop(0, T // R, chunk_body, jnp.zeros((1, E), jnp.float32))

    tokf = tok.astype(jnp.float32)
    iota_c = lax.broadcasted_iota(jnp.int32, (1, C), 1).astype(jnp.float32)
    for e in range(E):
        pcol = p_ref[:, e : e + 1]
        mcol = m_ref[:, e : e + 1]
        gcol = g_ref[:, e : e + 1]
        key = jnp.where(mcol > 0, pcol, jnp.float32(C))
        oh = (key == iota_c).astype(jnp.bfloat16)
        a = jnp.concatenate([tokf, gcol], axis=1).astype(jnp.bfloat16)
        res = lax.dot_general(
            oh, a, (((0,), (0,)), ((), ())),
            preferred_element_type=jnp.float32,
        )
        stok_ref[:, e : e + 1] = res[:, 0:1].astype(jnp.int32)
        gsel_ref[:, e : e + 1] = res[:, 1:2]

    p = p_ref[...]
    ps1 = jnp.sum(jnp.where(oh1 > 0, p, 0.0), axis=-1, keepdims=True)
    ps2 = jnp.sum(jnp.where(oh2 > 0, p, 0.0), axis=-1, keepdims=True)
    ok1 = ps1 < C
    ok2 = ps2 < C
    cidx_ref[:, 0:1] = jnp.where(ok1, i1 * C + ps1, 0.0).astype(jnp.int32)
    cidx_ref[:, 1:2] = jnp.where(ok2, i2 * C + ps2, 0.0).astype(jnp.int32)
    cmask_ref[:, 0:1] = ok1.astype(jnp.float32)
    cmask_ref[:, 1:2] = ok2.astype(jnp.float32)


def _router(tok2, embed, wr, br2, wlm, blm2):
    return pl.pallas_call(
        _router_body,
        out_shape=(
            jax.ShapeDtypeStruct((C, E), jnp.int32),
            jax.ShapeDtypeStruct((C, E), jnp.float32),
            jax.ShapeDtypeStruct((T, 2), jnp.int32),
            jax.ShapeDtypeStruct((T, 2), jnp.float32),
            jax.ShapeDtypeStruct((T, V), jnp.float32),
        ),
        scratch_shapes=[
            pltpu.VMEM((T, E), jnp.float32),
            pltpu.VMEM((T, E), jnp.float32),
            pltpu.VMEM((T, E), jnp.float32),
        ],
    )(tok2, embed, wr, br2, wlm, blm2)


def _ffn_body(stok_ref, emb_ref, w1_ref, b1_ref, w2_ref, b2_ref, g_ref,
              wlm_ref, zg_ref):
    iota_v = lax.broadcasted_iota(jnp.int32, (1, V), 1)
    onehot = (stok_ref[...] == iota_v).astype(jnp.float32)
    xg = jnp.dot(onehot, emb_ref[...], preferred_element_type=jnp.float32)
    h = jnp.dot(xg, w1_ref[0], preferred_element_type=jnp.float32)
    h = jax.nn.gelu(h + b1_ref[0])
    y = jnp.dot(h, w2_ref[0], preferred_element_type=jnp.float32)
    ye = (y + b2_ref[0]) * g_ref[...]
    z = jnp.dot(ye, wlm_ref[...], preferred_element_type=jnp.float32)
    zg_ref[...] = jnp.concatenate(
        [z, jnp.zeros((C, VP - V), jnp.float32)], axis=1
    )


def _ffn(stokcol, emb, w1, b1, w2, b2, gcolv, wlm):
    return pl.pallas_call(
        _ffn_body,
        grid=(E,),
        in_specs=[
            pl.BlockSpec((C, 1), lambda e: (e, 0)),
            pl.BlockSpec((V, D), lambda e: (0, 0)),
            pl.BlockSpec((1, D, F), lambda e: (e, 0, 0)),
            pl.BlockSpec((1, 1, F), lambda e: (e, 0, 0)),
            pl.BlockSpec((1, F, D), lambda e: (e, 0, 0)),
            pl.BlockSpec((1, 1, D), lambda e: (e, 0, 0)),
            pl.BlockSpec((C, 1), lambda e: (e, 0)),
            pl.BlockSpec((D, V), lambda e: (0, 0)),
        ],
        out_specs=pl.BlockSpec((C, VP), lambda e: (e, 0)),
        out_shape=jax.ShapeDtypeStruct((E * C, VP), jnp.float32),
    )(stokcol, emb, w1, b1.reshape(E, 1, F), w2, b2.reshape(E, 1, D),
      gcolv, wlm)


def _combine_body(xlm_ref, z0_ref, z1_ref, cm_ref, out_ref):
    cm = cm_ref[...]
    out_ref[...] = (
        xlm_ref[...]
        + z0_ref[:, :V] * cm[:, 0:1]
        + z1_ref[:, :V] * cm[:, 1:2]
    )


def _combine(xlm, z01, cmask):
    return pl.pallas_call(
        _combine_body,
        grid=(1,),
        in_specs=[
            pl.BlockSpec((T, V), lambda r: (0, 0)),
            pl.BlockSpec((T, VP), lambda r: (0, 0)),
            pl.BlockSpec((T, VP), lambda r: (1, 0)),
            pl.BlockSpec((T, 2), lambda r: (0, 0)),
        ],
        out_specs=pl.BlockSpec((T, V), lambda r: (0, 0)),
        out_shape=jax.ShapeDtypeStruct((T, V), jnp.float32),
    )(xlm, z01, z01, cmask)


def _lazy(maker):
    cache = []

    def call(table, idx):
        if not cache:
            cache.append(maker())
        return cache[0](table, idx)

    return call


_combine_gather = _lazy(lambda: _make_sc_gather((VP,), jnp.float32, 2 * T, 64))


def kernel(input_tensor, embed, Wr, br, W1, b1, W2, b2, Wlm, blm):
    tok = input_tensor.astype(jnp.int32)
    stok, gsel, cidx, cmask, xlm = _router(
        tok.reshape(T, 1), embed, Wr, br.reshape(1, E), Wlm, blm.reshape(1, V)
    )
    zg = _ffn(stok.T.reshape(E * C, 1), embed, W1, b1, W2, b2,
              gsel.T.reshape(E * C, 1), Wlm)
    z01 = _combine_gather(zg, cidx.T.reshape(2 * T))
    return _combine(xlm, z01, cmask)
